# baseline (speedup 1.0000x reference)
import jax
import jax.numpy as jnp
from jax.experimental import pallas as pl
from jax.experimental.pallas import tpu as pltpu

_ALPHA = 0.4
_SUB = 8
_INTERPRET = False


def _mix_body(perm_ref, lam_ref, ap_ref, bp_ref, aq_ref, bq_ref, op_ref, oq_ref):
    i = pl.program_id(0)
    lam = jax.lax.bitcast_convert_type(lam_ref[i], jnp.float32)
    one_m = 1.0 - lam
    a = ap_ref[0]
    b = bp_ref[0]
    op_ref[0] = a + jnp.log(lam + one_m * jnp.exp(b - a))
    a = aq_ref[0]
    b = bq_ref[0]
    oq_ref[0] = a + jnp.log(lam + one_m * jnp.exp(b - a))


def kernel(x_pre, x_post):
    b, g = x_pre.shape
    lane = g // _SUB
    key = jax.random.key(1)
    kp, kl = jax.random.split(key)
    perm = jax.random.permutation(kp, b)
    lam = jax.random.beta(kl, _ALPHA, _ALPHA, (b,)).astype(jnp.float32)
    lam_bits = jax.lax.bitcast_convert_type(lam, jnp.int32)

    x_pre3 = x_pre.reshape(b, _SUB, lane)
    x_post3 = x_post.reshape(b, _SUB, lane)

    own = pl.BlockSpec((1, _SUB, lane), lambda i, perm_r, lam_r: (i, 0, 0))
    gat = pl.BlockSpec((1, _SUB, lane), lambda i, perm_r, lam_r: (perm_r[i], 0, 0))

    grid_spec = pltpu.PrefetchScalarGridSpec(
        num_scalar_prefetch=2,
        grid=(b,),
        in_specs=[own, gat, own, gat],
        out_specs=[own, own],
    )
    out_shape = [jax.ShapeDtypeStruct((b, _SUB, lane), jnp.float32)] * 2
    op3, oq3 = pl.pallas_call(
        _mix_body,
        grid_spec=grid_spec,
        out_shape=out_shape,
        interpret=_INTERPRET,
    )(perm, lam_bits, x_pre3, x_pre3, x_post3, x_post3)
    return op3.reshape(b, g), oq3.reshape(b, g), lam, perm


# speedup vs baseline: 2.8229x; 2.8229x over previous
import jax
import jax.numpy as jnp
from jax.experimental import pallas as pl
from jax.experimental.pallas import tpu as pltpu

_ALPHA = 0.4
_SUB = 8
_R = 8
_INTERPRET = False


def _mix_body(perm_ref, lam_ref, *refs):
    i = pl.program_id(0)
    ap_ref = refs[0]
    aq_ref = refs[1]
    gp = refs[2:2 + _R]
    gq = refs[2 + _R:2 + 2 * _R]
    op_ref = refs[2 + 2 * _R]
    oq_ref = refs[3 + 2 * _R]
    for r in range(_R):
        lam = jax.lax.bitcast_convert_type(lam_ref[i * _R + r], jnp.float32)
        one_m = 1.0 - lam
        a = ap_ref[r]
        b = gp[r][0]
        op_ref[r] = a + jnp.log(lam + one_m * jnp.exp(b - a))
        a = aq_ref[r]
        b = gq[r][0]
        oq_ref[r] = a + jnp.log(lam + one_m * jnp.exp(b - a))


def kernel(x_pre, x_post):
    b, g = x_pre.shape
    lane = g // _SUB
    key = jax.random.key(1)
    kp, kl = jax.random.split(key)
    perm = jax.random.permutation(kp, b)
    lam = jax.random.beta(kl, _ALPHA, _ALPHA, (b,)).astype(jnp.float32)
    lam_bits = jax.lax.bitcast_convert_type(lam, jnp.int32)

    x_pre3 = x_pre.reshape(b, _SUB, lane)
    x_post3 = x_post.reshape(b, _SUB, lane)

    own = pl.BlockSpec((_R, _SUB, lane), lambda i, perm_r, lam_r: (i, 0, 0))

    def gat_spec(r):
        return pl.BlockSpec(
            (1, _SUB, lane),
            lambda i, perm_r, lam_r: (perm_r[i * _R + r], 0, 0),
        )

    grid_spec = pltpu.PrefetchScalarGridSpec(
        num_scalar_prefetch=2,
        grid=(b // _R,),
        in_specs=[own, own]
        + [gat_spec(r) for r in range(_R)]
        + [gat_spec(r) for r in range(_R)],
        out_specs=[own, own],
    )
    out_shape = [jax.ShapeDtypeStruct((b, _SUB, lane), jnp.float32)] * 2
    op3, oq3 = pl.pallas_call(
        _mix_body,
        grid_spec=grid_spec,
        out_shape=out_shape,
        interpret=_INTERPRET,
    )(perm, lam_bits, x_pre3, x_post3,
      *([x_pre3] * _R), *([x_post3] * _R))
    return op3.reshape(b, g), oq3.reshape(b, g), lam, perm


# speedup vs baseline: 3.9884x; 1.4128x over previous
import jax
import jax.numpy as jnp
from jax import lax
from jax.experimental import pallas as pl
from jax.experimental.pallas import tpu as pltpu

_ALPHA = 0.4
_R = 8
_INTERPRET = False


def _mix_body(perm_ref, ap_ref, aq_ref, lam_ref, hp_ref, hq_ref,
              op_ref, oq_ref, bp0, bq0, bp1, bq1, semp, semq):
    i = pl.program_id(0)
    n = pl.num_programs(0)
    even = lax.rem(i, 2) == 0

    def issue(step, bufp, bufq):
        base = step * _R
        for j in range(_R):
            row = perm_ref[base + j]
            pltpu.make_async_copy(
                hp_ref.at[pl.ds(row, 1)], bufp.at[pl.ds(j, 1)], semp).start()
            pltpu.make_async_copy(
                hq_ref.at[pl.ds(row, 1)], bufq.at[pl.ds(j, 1)], semq).start()

    def drain(bufp, bufq):
        for j in range(_R):
            pltpu.make_async_copy(
                hp_ref.at[pl.ds(0, 1)], bufp.at[pl.ds(j, 1)], semp).wait()
            pltpu.make_async_copy(
                hq_ref.at[pl.ds(0, 1)], bufq.at[pl.ds(j, 1)], semq).wait()

    @pl.when(i == 0)
    def _():
        issue(0, bp0, bq0)

    @pl.when(jnp.logical_and(even, i + 1 < n))
    def _():
        issue(i + 1, bp1, bq1)

    @pl.when(jnp.logical_and(jnp.logical_not(even), i + 1 < n))
    def _():
        issue(i + 1, bp0, bq0)

    lam = lam_ref[...]
    one_m = 1.0 - lam

    def compute(bufp, bufq):
        drain(bufp, bufq)
        a = ap_ref[...]
        b = bufp[...]
        op_ref[...] = a + jnp.log(lam + one_m * jnp.exp(b - a))
        a = aq_ref[...]
        b = bufq[...]
        oq_ref[...] = a + jnp.log(lam + one_m * jnp.exp(b - a))

    @pl.when(even)
    def _():
        compute(bp0, bq0)

    @pl.when(jnp.logical_not(even))
    def _():
        compute(bp1, bq1)


def kernel(x_pre, x_post):
    b, g = x_pre.shape
    key = jax.random.key(1)
    kp, kl = jax.random.split(key)
    perm = jax.random.permutation(kp, b)
    lam = jax.random.beta(kl, _ALPHA, _ALPHA, (b,)).astype(jnp.float32)

    hbm = pl.BlockSpec(memory_space=pl.ANY)
    lam_spec = pl.BlockSpec((_R, 1), lambda i, perm_r: (i, 0))
    out_spec = pl.BlockSpec((_R, g), lambda i, perm_r: (i, 0))

    grid_spec = pltpu.PrefetchScalarGridSpec(
        num_scalar_prefetch=1,
        grid=(b // _R,),
        in_specs=[out_spec, out_spec, lam_spec, hbm, hbm],
        out_specs=[out_spec, out_spec],
        scratch_shapes=[
            pltpu.VMEM((_R, g), jnp.float32),
            pltpu.VMEM((_R, g), jnp.float32),
            pltpu.VMEM((_R, g), jnp.float32),
            pltpu.VMEM((_R, g), jnp.float32),
            pltpu.SemaphoreType.DMA,
            pltpu.SemaphoreType.DMA,
        ],
    )
    out_shape = [jax.ShapeDtypeStruct((b, g), jnp.float32)] * 2
    op, oq = pl.pallas_call(
        _mix_body,
        grid_spec=grid_spec,
        out_shape=out_shape,
        interpret=_INTERPRET,
    )(perm, x_pre, x_post, lam.reshape(b, 1), x_pre, x_post)
    return op, oq, lam, perm


# speedup vs baseline: 4.3919x; 1.1012x over previous
import jax
import jax.numpy as jnp
from jax import lax
from jax.experimental import pallas as pl
from jax.experimental.pallas import tpu as pltpu

_ALPHA = 0.4
_R = 8
_NBUF = 3
_INTERPRET = False


def _mix_body(perm_ref, ap_ref, aq_ref, lam_ref, hp_ref, hq_ref,
              op_ref, oq_ref, bufs_p, bufs_q, semp, semq):
    i = pl.program_id(0)
    n = pl.num_programs(0)
    rem = lax.rem(i, _NBUF)

    def issue(step, s):
        base = step * _R
        for j in range(_R):
            row = perm_ref[base + j]
            pltpu.make_async_copy(
                hp_ref.at[pl.ds(row, 1)], bufs_p[s].at[pl.ds(j, 1)],
                semp.at[s]).start()
            pltpu.make_async_copy(
                hq_ref.at[pl.ds(row, 1)], bufs_q[s].at[pl.ds(j, 1)],
                semq.at[s]).start()

    @pl.when(i == 0)
    def _():
        for k in range(_NBUF - 1):
            issue(k, k)

    lam = lam_ref[...]
    one_m = 1.0 - lam

    def step_for(s):
        s2 = (s + _NBUF - 1) % _NBUF

        @pl.when(i + _NBUF - 1 < n)
        def _():
            issue(i + _NBUF - 1, s2)

        for j in range(_R):
            pltpu.make_async_copy(
                hp_ref.at[pl.ds(0, 1)], bufs_p[s].at[pl.ds(j, 1)],
                semp.at[s]).wait()
            pltpu.make_async_copy(
                hq_ref.at[pl.ds(0, 1)], bufs_q[s].at[pl.ds(j, 1)],
                semq.at[s]).wait()
        a = ap_ref[...]
        b = bufs_p[s][...]
        op_ref[...] = a + jnp.log(lam + one_m * jnp.exp(b - a))
        a = aq_ref[...]
        b = bufs_q[s][...]
        oq_ref[...] = a + jnp.log(lam + one_m * jnp.exp(b - a))

    for s in range(_NBUF):
        @pl.when(rem == s)
        def _(s=s):
            step_for(s)


def kernel(x_pre, x_post):
    b, g = x_pre.shape
    key = jax.random.key(1)
    kp, kl = jax.random.split(key)
    perm = jax.random.permutation(kp, b)
    lam = jax.random.beta(kl, _ALPHA, _ALPHA, (b,)).astype(jnp.float32)

    hbm = pl.BlockSpec(memory_space=pl.ANY)
    lam_spec = pl.BlockSpec((_R, 1), lambda i, perm_r: (i, 0))
    out_spec = pl.BlockSpec((_R, g), lambda i, perm_r: (i, 0))

    grid_spec = pltpu.PrefetchScalarGridSpec(
        num_scalar_prefetch=1,
        grid=(b // _R,),
        in_specs=[out_spec, out_spec, lam_spec, hbm, hbm],
        out_specs=[out_spec, out_spec],
        scratch_shapes=[
            [pltpu.VMEM((_R, g), jnp.float32) for _ in range(_NBUF)],
            [pltpu.VMEM((_R, g), jnp.float32) for _ in range(_NBUF)],
            pltpu.SemaphoreType.DMA((_NBUF,)),
            pltpu.SemaphoreType.DMA((_NBUF,)),
        ],
    )
    out_shape = [jax.ShapeDtypeStruct((b, g), jnp.float32)] * 2
    op, oq = pl.pallas_call(
        _mix_body,
        grid_spec=grid_spec,
        out_shape=out_shape,
        interpret=_INTERPRET,
    )(perm, x_pre, x_post, lam.reshape(b, 1), x_pre, x_post)
    return op, oq, lam, perm
